# baseline (speedup 1.0000x reference)
import functools

import jax
import jax.numpy as jnp
from jax import lax
from jax.experimental import pallas as pl
from jax.experimental.pallas import tpu as pltpu
from jax.experimental.pallas import tpu_sc as plsc

NC, NS, L = 2, 16, 16
NW = NC * NS

N_NODE = 8000
TBL = 8064
RPS = TBL // NS
DF = 128
DW = 16
DO = 64

E = 120000
ECH = 128
NCH = 32
EPAD = NW * NCH * ECH
DUMMY = 8000

NLBL = 200000
LCH = 128
NLCH = 56
LPAD = NW * NLCH * LCH

ROWBLK = 1000


def _agg_body(table, srcr, dstr, z128,
              agg_out,
              sidx, didx, buf0, stable):
  c = lax.axis_index("c")
  s = lax.axis_index("s")
  wid = c * NS + s

  @pl.when(s == 0)
  def _init():
    pltpu.sync_copy(z128, stable)

  pltpu.sync_copy(srcr.at[wid], sidx)
  pltpu.sync_copy(dstr.at[wid], didx)
  plsc.subcore_barrier()

  @pl.loop(0, NCH)
  def _chunk(ch):
    pltpu.sync_copy(table.at[sidx.at[ch]], buf0)
    pltpu.sync_copy(buf0, stable.at[didx.at[ch]], add=True)

  plsc.subcore_barrier()
  @pl.when(s == 0)
  def _out():
    pltpu.sync_copy(stable, agg_out.at[c])


def _sc_agg(table, srcr, dstr, z128):
  mesh = plsc.VectorSubcoreMesh(core_axis_name="c", subcore_axis_name="s",
                                num_cores=NC, num_subcores=NS)
  return pl.kernel(
      _agg_body,
      out_type=jax.ShapeDtypeStruct((NC, TBL, DF), jnp.float32),
      mesh=mesh,
      scratch_types=[
          pltpu.VMEM((NCH, ECH), jnp.int32),
          pltpu.VMEM((NCH, ECH), jnp.int32),
          pltpu.VMEM((ECH, DF), jnp.float32),
          pltpu.VMEM_SHARED((TBL, DF), jnp.float32),
      ],
      name="sc_segment_sum",
  )(table, srcr, dstr, z128)


def _deg_body(dstr, z128, ones_h,
              deg_out,
              didx, ones_v, dtable):
  c = lax.axis_index("c")
  s = lax.axis_index("s")
  wid = c * NS + s

  @pl.when(s == 0)
  def _init():
    pltpu.sync_copy(z128, dtable)

  pltpu.sync_copy(dstr.at[wid], didx)
  pltpu.sync_copy(ones_h, ones_v)
  plsc.subcore_barrier()

  @pl.loop(0, NCH)
  def _chunk(ch):
    pltpu.sync_copy(ones_v, dtable.at[didx.at[ch]], add=True)

  plsc.subcore_barrier()
  @pl.when(s == 0)
  def _out():
    pltpu.sync_copy(dtable, deg_out.at[c])


def _sc_deg(dstr, z128, ones_h):
  mesh = plsc.VectorSubcoreMesh(core_axis_name="c", subcore_axis_name="s",
                                num_cores=NC, num_subcores=NS)
  return pl.kernel(
      _deg_body,
      out_type=jax.ShapeDtypeStruct((NC, TBL, DF), jnp.float32),
      mesh=mesh,
      scratch_types=[
          pltpu.VMEM((NCH, ECH), jnp.int32),
          pltpu.VMEM((ECH, DF), jnp.float32),
          pltpu.VMEM_SHARED((TBL, DF), jnp.float32),
      ],
      name="sc_degrees",
  )(dstr, z128, ones_h)


def _gather_body(tab, ar, br, outa, outb,
                 aidx, bidx, bufa0, bufb0):
  c = lax.axis_index("c")
  s = lax.axis_index("s")
  wid = c * NS + s

  pltpu.sync_copy(ar.at[wid], aidx)
  pltpu.sync_copy(br.at[wid], bidx)

  @pl.loop(0, NLCH)
  def _chunk(ch):
    pltpu.sync_copy(tab.at[aidx.at[ch]], bufa0)
    pltpu.sync_copy(tab.at[bidx.at[ch]], bufb0)
    pltpu.sync_copy(bufa0, outa.at[wid * NLCH + ch])
    pltpu.sync_copy(bufb0, outb.at[wid * NLCH + ch])


def _sc_gather_pairs(tab, ar, br):
  mesh = plsc.VectorSubcoreMesh(core_axis_name="c", subcore_axis_name="s",
                                num_cores=NC, num_subcores=NS)
  return pl.kernel(
      _gather_body,
      out_type=[
          jax.ShapeDtypeStruct((NW * NLCH, LCH, 2 * DO), jnp.float32),
          jax.ShapeDtypeStruct((NW * NLCH, LCH, 2 * DO), jnp.float32),
      ],
      mesh=mesh,
      scratch_types=[
          pltpu.VMEM((NLCH, LCH), jnp.int32),
          pltpu.VMEM((NLCH, LCH), jnp.int32),
          pltpu.VMEM((LCH, 2 * DO), jnp.float32),
          pltpu.VMEM((LCH, 2 * DO), jnp.float32),
      ],
      name="sc_gather_pairs",
  )(tab, ar, br)


DBLK = 64


def _tcdec_body(a, b, o):
  o[...] = jnp.sum(a[..., :DO] * b[..., DO:], axis=-1)


def _tc_decode(ga, gb):
  nblk = (NW * NLCH) // DBLK
  return pl.pallas_call(
      _tcdec_body,
      grid=(nblk,),
      in_specs=[
          pl.BlockSpec((DBLK, LCH, 2 * DO), lambda i: (i, 0, 0)),
          pl.BlockSpec((DBLK, LCH, 2 * DO), lambda i: (i, 0, 0)),
      ],
      out_specs=pl.BlockSpec((DBLK, LCH), lambda i: (i, 0)),
      out_shape=jax.ShapeDtypeStruct((NW * NLCH, LCH), jnp.float32),
      name="tc_distmult_decode",
  )(ga, gb)


def _tc1_body(xc, xd, aggp, degp, wsc, wsd, wr, h0c, h0d):
  hc = jnp.dot(xc[...], wsc[...], preferred_element_type=jnp.float32)
  h0c[...] = jnp.maximum(hc, 0.0)
  agg = aggp[0] + aggp[1]
  deg = degp[0, :, 0] + degp[1, :, 0]
  m = agg * (1.0 / jnp.maximum(deg, 1.0))[:, None]
  hd = (jnp.dot(xd[...], wsd[...], preferred_element_type=jnp.float32)
        + jnp.dot(m, wr[...], preferred_element_type=jnp.float32))
  h0d[...] = jnp.maximum(hd, 0.0)


def _tc_layer0(xc, xd, aggp, degp, wsc, wsd, wr):
  nblk = N_NODE // ROWBLK
  return pl.pallas_call(
      _tc1_body,
      grid=(nblk,),
      in_specs=[
          pl.BlockSpec((ROWBLK, DF), lambda i: (i, 0)),
          pl.BlockSpec((ROWBLK, DF), lambda i: (i, 0)),
          pl.BlockSpec((NC, ROWBLK, DF), lambda i: (0, i, 0)),
          pl.BlockSpec((NC, ROWBLK, DF), lambda i: (0, i, 0)),
          pl.BlockSpec((DF, DF), lambda i: (0, 0)),
          pl.BlockSpec((DF, DF), lambda i: (0, 0)),
          pl.BlockSpec((DF, DF), lambda i: (0, 0)),
      ],
      out_specs=[
          pl.BlockSpec((ROWBLK, DF), lambda i: (i, 0)),
          pl.BlockSpec((ROWBLK, DF), lambda i: (i, 0)),
      ],
      out_shape=[
          jax.ShapeDtypeStruct((N_NODE, DF), jnp.float32),
          jax.ShapeDtypeStruct((N_NODE, DF), jnp.float32),
      ],
      name="tc_rgcn_layer0",
  )(xc, xd, aggp, degp, wsc, wsd, wr)


def _tc2_body(hc, hd, aggp, degp, wsc, wsd, wr, rv, tab):
  gc = jnp.dot(hc[...], wsc[...], preferred_element_type=jnp.float32) * rv[...]
  agg = aggp[0] + aggp[1]
  deg = degp[0, :, 0] + degp[1, :, 0]
  m = agg * (1.0 / jnp.maximum(deg, 1.0))[:, None]
  h1d = (jnp.dot(hd[...], wsd[...], preferred_element_type=jnp.float32)
         + jnp.dot(m, wr[...], preferred_element_type=jnp.float32))
  tab[...] = jnp.concatenate([gc, h1d], axis=1)


def _tc_layer1(hc, hd, aggp, degp, wsc, wsd, wr, rv):
  nblk = N_NODE // ROWBLK
  return pl.pallas_call(
      _tc2_body,
      grid=(nblk,),
      in_specs=[
          pl.BlockSpec((ROWBLK, DF), lambda i: (i, 0)),
          pl.BlockSpec((ROWBLK, DF), lambda i: (i, 0)),
          pl.BlockSpec((NC, ROWBLK, DF), lambda i: (0, i, 0)),
          pl.BlockSpec((NC, ROWBLK, DF), lambda i: (0, i, 0)),
          pl.BlockSpec((DF, DO), lambda i: (0, 0)),
          pl.BlockSpec((DF, DO), lambda i: (0, 0)),
          pl.BlockSpec((DF, DO), lambda i: (0, 0)),
          pl.BlockSpec((1, DO), lambda i: (0, 0)),
      ],
      out_specs=pl.BlockSpec((ROWBLK, 2 * DO), lambda i: (i, 0)),
      out_shape=jax.ShapeDtypeStruct((N_NODE, 2 * DO), jnp.float32),
      name="tc_rgcn_layer1",
  )(hc, hd, aggp, degp, wsc, wsd, wr, rv)


def kernel(x_Compound, x_Disease, x_Gene, x_Anatomy,
           edge_index_0, edge_index_1, edge_index_2, edge_index_3,
           W_self_0_Compound, W_self_0_Disease, W_self_0_Gene, W_self_0_Anatomy,
           W_rel_0_0, W_rel_0_1, W_rel_0_2, W_rel_0_3,
           W_self_1_Compound, W_self_1_Disease, W_self_1_Gene, W_self_1_Anatomy,
           W_rel_1_0, W_rel_1_1, W_rel_1_2, W_rel_1_3,
           rel_vec, edge_label_index):
  src = jnp.concatenate(
      [edge_index_0[0].astype(jnp.int32), jnp.zeros((EPAD - E,), jnp.int32)])
  dst = jnp.concatenate(
      [edge_index_0[1].astype(jnp.int32),
       jnp.full((EPAD - E,), DUMMY, jnp.int32)])
  srcr = src.reshape(NW, NCH, ECH)
  dstr = dst.reshape(NW, NCH, ECH)

  z128 = jnp.zeros((TBL, DF), jnp.float32)
  ones_h = jnp.ones((ECH, DF), jnp.float32)

  degp = _sc_deg(dstr, z128, ones_h)
  aggx = _sc_agg(x_Compound, srcr, dstr, z128)
  h0c, h0d = _tc_layer0(x_Compound, x_Disease, aggx, degp,
                        W_self_0_Compound, W_self_0_Disease, W_rel_0_0)
  aggh = _sc_agg(h0c, srcr, dstr, z128)
  tab = _tc_layer1(h0c, h0d, aggh, degp,
                   W_self_1_Compound, W_self_1_Disease, W_rel_1_0,
                   rel_vec.reshape(1, DO))

  a = jnp.concatenate(
      [edge_label_index[0].astype(jnp.int32),
       jnp.zeros((LPAD - NLBL,), jnp.int32)])
  b = jnp.concatenate(
      [edge_label_index[1].astype(jnp.int32),
       jnp.zeros((LPAD - NLBL,), jnp.int32)])
  ar = a.reshape(NW, NLCH, LCH)
  br = b.reshape(NW, NLCH, LCH)
  ga, gb = _sc_gather_pairs(tab, ar, br)
  scores = _tc_decode(ga, gb)
  return scores.reshape(LPAD)[:NLBL]
